# baseline (speedup 1.0000x reference)
import math

import jax
import jax.numpy as jnp
from jax.experimental import pallas as pl
from jax.experimental.pallas import tpu as pltpu

H = 8
D = 256
DK = D // H
B = 2
N = 2048
M = 4096

TM = 256
MT = M // TM
NT = N // MT

_INV_DK = 1.0 / DK
_INV_SQRT_DK = 1.0 / math.sqrt(DK)
_LN_EPS = 1e-5
_HODGE_EPS = 1e-6


_SQRT_DK = math.sqrt(DK)


def _group_sum(x, a, precision):
    return jax.lax.dot_general(x, a, (((1,), (0,)), ((), ())),
                               preferred_element_type=jnp.float32,
                               precision=precision)


def _ln_pair(x1, x2, a, g1, b1, g2, b2, precision):
    r = x1.shape[0]
    mu = _group_sum(jnp.concatenate([x1, x2], axis=0), a, precision) * _INV_DK
    c1 = x1 - mu[:r]
    c2 = x2 - mu[r:]
    var = _group_sum(jnp.concatenate([c1 * c1, c2 * c2], axis=0), a,
                     precision) * _INV_DK
    y1 = (c1 / jnp.sqrt(var[:r] + _LN_EPS)) * g1 + b1
    y2 = (c2 / jnp.sqrt(var[r:] + _LN_EPS)) * g2 + b2
    return y1, y2


def _body(gb_ref, wq_ref, wk_ref, weq_ref, wek_ref, wo_ref, a_ref,
          v_ref, e_ref, d0_ref, out_ref, acc_ref, hv_ref):
    mi = pl.program_id(1)
    a = a_ref[...]
    vb = v_ref[0]

    @pl.when(mi == 0)
    def _init():
        acc_ref[...] = jnp.zeros_like(acc_ref)

    hi = jax.lax.Precision.HIGHEST
    vs = v_ref[0, pl.ds(mi * NT, NT), :]
    q = jnp.dot(vs, wq_ref[...], preferred_element_type=jnp.float32)
    k = jnp.dot(vs, wk_ref[...], preferred_element_type=jnp.float32)
    q, k = _ln_pair(q, k, a, gb_ref[0:1, :], gb_ref[1:2, :],
                    gb_ref[2:3, :], gb_ref[3:4, :], hi)
    hvc = _group_sum(q * k, a, hi) / _SQRT_DK
    hv_ref[pl.ds(mi * NT, NT), :] = 1.0 / (hvc + _HODGE_EPS)

    eb = e_ref[0]
    eq = jnp.dot(eb, weq_ref[...], preferred_element_type=jnp.float32)
    ek = jnp.dot(eb, wek_ref[...], preferred_element_type=jnp.float32)
    eq, ek = _ln_pair(eq, ek, a, gb_ref[4:5, :], gb_ref[5:6, :],
                      gb_ref[6:7, :], gb_ref[7:8, :], None)
    he = _group_sum(eq * ek, a, None) / _SQRT_DK

    d0b = d0_ref[0].astype(jnp.bfloat16)
    vb16 = vb.astype(jnp.bfloat16)
    x1 = jnp.dot(d0b, vb16, preferred_element_type=jnp.float32) * he
    acc_ref[...] += jax.lax.dot_general(
        d0b, x1.astype(jnp.bfloat16), (((0,), (0,)), ((), ())),
        preferred_element_type=jnp.float32)

    @pl.when(mi == MT - 1)
    def _fin():
        out_ref[0] = jnp.dot(acc_ref[...] * hv_ref[...], wo_ref[...],
                             preferred_element_type=jnp.float32)


def kernel(v, e, d_0, v_idx, e_idx, W_vQ, W_vK, W_vO, W_eQ, W_eK,
           g_vq, b_vq, g_vk, b_vk, g_eq, b_eq, g_ek, b_ek):
    del v_idx, e_idx
    f32 = jnp.float32
    idx = jnp.arange(D)
    a = (idx[:, None] // DK == idx[None, :] // DK).astype(f32)

    wq = W_vQ.T
    wk = W_vK.T
    weq = W_eQ.T
    wek = W_eK.T
    wo = W_vO.T
    gb = jnp.concatenate([
        g_vq.reshape(1, D), b_vq.reshape(1, D),
        g_vk.reshape(1, D), b_vk.reshape(1, D),
        g_eq.reshape(1, D), b_eq.reshape(1, D),
        g_ek.reshape(1, D), b_ek.reshape(1, D)], axis=0)

    full = lambda shape: pl.BlockSpec(shape, lambda b_, m_: (0,) * len(shape))
    out = pl.pallas_call(
        _body,
        grid=(B, MT),
        in_specs=[
            full((8, D)),
            full((D, D)),
            full((D, D)),
            full((D, D)),
            full((D, D)),
            full((D, D)),
            full((D, D)),
            pl.BlockSpec((1, N, D), lambda b_, m_: (b_, 0, 0)),
            pl.BlockSpec((1, TM, D), lambda b_, m_: (b_, m_, 0)),
            pl.BlockSpec((1, TM, N), lambda b_, m_: (b_, m_, 0)),
        ],
        out_specs=pl.BlockSpec((1, N, D), lambda b_, m_: (b_, 0, 0)),
        out_shape=jax.ShapeDtypeStruct((B, N, D), f32),
        scratch_shapes=[
            pltpu.VMEM((N, D), f32),
            pltpu.VMEM((N, D), f32),
        ],
        compiler_params=pltpu.CompilerParams(
            dimension_semantics=("arbitrary", "arbitrary"),
        ),
    )(gb, wq, wk, weq, wek, wo, a, v, e, d_0)
    return out


# speedup vs baseline: 1.1545x; 1.1545x over previous
import math

import jax
import jax.numpy as jnp
from jax.experimental import pallas as pl
from jax.experimental.pallas import tpu as pltpu

H = 8
D = 256
DK = D // H
B = 2
N = 2048
M = 4096

TM = 256
MT = M // TM

_INV_DK = 1.0 / DK
_SQRT_DK = math.sqrt(DK)
_LN_EPS = 1e-5
_HODGE_EPS = 1e-6


def _group_sum(x, a, precision):
    return jax.lax.dot_general(x, a, (((1,), (0,)), ((), ())),
                               preferred_element_type=jnp.float32,
                               precision=precision)


def _ln_faithful(x, a, g, b, precision):
    mu = _group_sum(x, a, precision) * _INV_DK
    xc = x - mu
    var = _group_sum(xc * xc, a, precision) * _INV_DK
    return (xc / jnp.sqrt(var + _LN_EPS)) * g + b


def _body(gb_ref, wq_ref, wk_ref, weq_ref, wek_ref, wo_ref, a_ref,
          v_ref, e_ref, d0_ref, out_ref, acc_ref, hv_ref, he_ref):
    mi = pl.program_id(1)

    @pl.when(mi == 0)
    def _stats():
        hi = jax.lax.Precision.HIGHEST
        a = a_ref[...]
        vb = v_ref[0]
        q = jnp.dot(vb, wq_ref[...], preferred_element_type=jnp.float32)
        k = jnp.dot(vb, wk_ref[...], preferred_element_type=jnp.float32)
        q = _ln_faithful(q, a, gb_ref[0:1, :], gb_ref[1:2, :], hi)
        k = _ln_faithful(k, a, gb_ref[2:3, :], gb_ref[3:4, :], hi)
        hvc = _group_sum(q * k, a, hi) / _SQRT_DK
        hv_ref[...] = 1.0 / (hvc + _HODGE_EPS)
        eb = e_ref[0]
        eq = jnp.dot(eb, weq_ref[...], preferred_element_type=jnp.float32)
        ek = jnp.dot(eb, wek_ref[...], preferred_element_type=jnp.float32)
        eq = _ln_faithful(eq, a, gb_ref[4:5, :], gb_ref[5:6, :], None)
        ek = _ln_faithful(ek, a, gb_ref[6:7, :], gb_ref[7:8, :], None)
        he_ref[...] = _group_sum(eq * ek, a, None) / _SQRT_DK
        acc_ref[...] = jnp.zeros_like(acc_ref)

    d0b = d0_ref[0].astype(jnp.bfloat16)
    x1 = jnp.dot(d0b, v_ref[0].astype(jnp.bfloat16),
                 preferred_element_type=jnp.float32)
    x1 = x1 * he_ref[pl.ds(mi * TM, TM), :]
    acc_ref[...] += jax.lax.dot_general(
        d0b, x1.astype(jnp.bfloat16), (((0,), (0,)), ((), ())),
        preferred_element_type=jnp.float32)

    @pl.when(mi == MT - 1)
    def _fin():
        out_ref[0] = jnp.dot(acc_ref[...] * hv_ref[...], wo_ref[...],
                             preferred_element_type=jnp.float32)


def kernel(v, e, d_0, v_idx, e_idx, W_vQ, W_vK, W_vO, W_eQ, W_eK,
           g_vq, b_vq, g_vk, b_vk, g_eq, b_eq, g_ek, b_ek):
    del v_idx, e_idx
    f32 = jnp.float32
    idx = jnp.arange(D)
    a = (idx[:, None] // DK == idx[None, :] // DK).astype(f32)

    wq = W_vQ.T
    wk = W_vK.T
    weq = W_eQ.T
    wek = W_eK.T
    wo = W_vO.T
    gb = jnp.concatenate([
        g_vq.reshape(1, D), b_vq.reshape(1, D),
        g_vk.reshape(1, D), b_vk.reshape(1, D),
        g_eq.reshape(1, D), b_eq.reshape(1, D),
        g_ek.reshape(1, D), b_ek.reshape(1, D)], axis=0)

    full = lambda shape: pl.BlockSpec(shape, lambda b_, m_: (0,) * len(shape))
    out = pl.pallas_call(
        _body,
        grid=(B, MT),
        in_specs=[
            full((8, D)),
            full((D, D)),
            full((D, D)),
            full((D, D)),
            full((D, D)),
            full((D, D)),
            full((D, D)),
            pl.BlockSpec((1, N, D), lambda b_, m_: (b_, 0, 0)),
            pl.BlockSpec((1, M, D), lambda b_, m_: (b_, 0, 0)),
            pl.BlockSpec((1, TM, N), lambda b_, m_: (b_, m_, 0)),
        ],
        out_specs=pl.BlockSpec((1, N, D), lambda b_, m_: (b_, 0, 0)),
        out_shape=jax.ShapeDtypeStruct((B, N, D), f32),
        scratch_shapes=[
            pltpu.VMEM((N, D), f32),
            pltpu.VMEM((N, D), f32),
            pltpu.VMEM((M, D), f32),
        ],
        compiler_params=pltpu.CompilerParams(
            dimension_semantics=("arbitrary", "arbitrary"),
        ),
    )(gb, wq, wk, weq, wek, wo, a, v, e, d_0)
    return out


# speedup vs baseline: 1.1562x; 1.0014x over previous
import math

import jax
import jax.numpy as jnp
from jax.experimental import pallas as pl
from jax.experimental.pallas import tpu as pltpu

H = 8
D = 256
DK = D // H
B = 2
N = 2048
M = 4096

TM = 256
MT = M // TM

_INV_DK = 1.0 / DK
_SQRT_DK = math.sqrt(DK)
_LN_EPS = 1e-5
_HODGE_EPS = 1e-6


def _group_sum(x, a, precision):
    return jax.lax.dot_general(x, a, (((1,), (0,)), ((), ())),
                               preferred_element_type=jnp.float32,
                               precision=precision)


def _ln_faithful(x, a, g, b, precision):
    mu = _group_sum(x, a, precision) * _INV_DK
    xc = x - mu
    var = _group_sum(xc * xc, a, precision) * _INV_DK
    return (xc / jnp.sqrt(var + _LN_EPS)) * g + b


def _body(gb_ref, wq_ref, wk_ref, weq_ref, wek_ref, wo_ref, a_ref,
          v_ref, e_ref, d0_ref, out_ref, acc_ref, hv_ref, he_ref):
    mi = pl.program_id(1)

    @pl.when(mi == 0)
    def _stats():
        hi = jax.lax.Precision.HIGHEST
        a = a_ref[...]
        vb = v_ref[0]
        q = jnp.dot(vb, wq_ref[...], preferred_element_type=jnp.float32)
        k = jnp.dot(vb, wk_ref[...], preferred_element_type=jnp.float32)
        q = _ln_faithful(q, a, gb_ref[0:1, :], gb_ref[1:2, :], hi)
        k = _ln_faithful(k, a, gb_ref[2:3, :], gb_ref[3:4, :], hi)
        hvc = _group_sum(q * k, a, hi) / _SQRT_DK
        hv_ref[...] = 1.0 / (hvc + _HODGE_EPS)
        eb = e_ref[0]
        eq = jnp.dot(eb, weq_ref[...], preferred_element_type=jnp.float32)
        ek = jnp.dot(eb, wek_ref[...], preferred_element_type=jnp.float32)
        eq = _ln_faithful(eq, a, gb_ref[4:5, :], gb_ref[5:6, :], None)
        ek = _ln_faithful(ek, a, gb_ref[6:7, :], gb_ref[7:8, :], None)
        he_ref[...] = _group_sum(eq * ek, a, None) / _SQRT_DK
        acc_ref[...] = jnp.zeros_like(acc_ref)

    d0b = d0_ref[0].astype(jnp.bfloat16)
    x1 = jnp.dot(d0b, v_ref[0].astype(jnp.bfloat16),
                 preferred_element_type=jnp.float32)
    x1 = x1 * he_ref[pl.ds(mi * TM, TM), :]
    acc_ref[...] += jax.lax.dot_general(
        d0b, x1.astype(jnp.bfloat16), (((0,), (0,)), ((), ())),
        preferred_element_type=jnp.float32)

    @pl.when(mi == MT - 1)
    def _fin():
        out_ref[0] = jnp.dot(acc_ref[...] * hv_ref[...], wo_ref[...],
                             preferred_element_type=jnp.float32)


def kernel(v, e, d_0, v_idx, e_idx, W_vQ, W_vK, W_vO, W_eQ, W_eK,
           g_vq, b_vq, g_vk, b_vk, g_eq, b_eq, g_ek, b_ek):
    del v_idx, e_idx
    f32 = jnp.float32
    idx = jnp.arange(D)
    a = (idx[:, None] // DK == idx[None, :] // DK).astype(f32)

    wq = W_vQ.T
    wk = W_vK.T
    weq = W_eQ.T
    wek = W_eK.T
    wo = W_vO.T
    gb = jnp.concatenate([
        g_vq.reshape(1, D), b_vq.reshape(1, D),
        g_vk.reshape(1, D), b_vk.reshape(1, D),
        g_eq.reshape(1, D), b_eq.reshape(1, D),
        g_ek.reshape(1, D), b_ek.reshape(1, D)], axis=0)

    full = lambda shape: pl.BlockSpec(shape, lambda b_, m_: (0,) * len(shape))
    out = pl.pallas_call(
        _body,
        grid=(B, MT),
        in_specs=[
            full((8, D)),
            full((D, D)),
            full((D, D)),
            full((D, D)),
            full((D, D)),
            full((D, D)),
            full((D, D)),
            pl.BlockSpec((1, N, D), lambda b_, m_: (b_, 0, 0)),
            pl.BlockSpec((1, M, D), lambda b_, m_: (b_, 0, 0)),
            pl.BlockSpec((1, TM, N), lambda b_, m_: (b_, m_, 0)),
        ],
        out_specs=pl.BlockSpec((1, N, D), lambda b_, m_: (b_, 0, 0)),
        out_shape=jax.ShapeDtypeStruct((B, N, D), f32),
        scratch_shapes=[
            pltpu.VMEM((N, D), f32),
            pltpu.VMEM((N, D), f32),
            pltpu.VMEM((M, D), f32),
        ],
        compiler_params=pltpu.CompilerParams(
            dimension_semantics=("parallel", "arbitrary"),
        ),
    )(gb, wq, wk, weq, wek, wo, a, v, e, d_0)
    return out
